# baseline (speedup 1.0000x reference)
import functools

import jax
import jax.numpy as jnp
from jax import lax
from jax.experimental import pallas as pl
from jax.experimental.pallas import tpu as pltpu
from jax.experimental.pallas import tpu_sc as plsc

EPS = 1e-7

_N = 10000
_E = 320000
_D = 128
_DH = 64
_C = 80
_NCHUNK = _E // _C
_NS = 16
_CPT = _NCHUNK // _NS
_NBUF = 2
_IBUF = 4
_RPT = _N // _NS
_RB = 25


def _sc_agg_body(x0_hbm, x1_hbm, attr_hbm, src_hbm, dst_hbm,
                 out0_hbm, out1_hbm,
                 src_t, dst_t, gx, at, pw, stg, acc_sh,
                 ssem, dsem, gsem, asem, scsem):
    c = lax.axis_index("c")
    s = lax.axis_index("s")

    @pl.loop(0, _RB)
    def _(i):
        for q in range(_D // 16):
            stg[i, pl.ds(q * 16, 16)] = jnp.zeros((16,), jnp.float32)

    rbase = s * _RPT
    for k in range(_RPT // _RB):
        pltpu.sync_copy(stg, acc_sh.at[pl.ds(rbase + k * _RB, _RB)])
    plsc.subcore_barrier()

    cstart = s * _CPT

    def issue_src(k, slot):
        pltpu.async_copy(src_hbm.at[cstart + k], src_t.at[slot], ssem.at[slot])

    def wait_src(slot):
        pltpu.make_async_copy(src_hbm.at[0], src_t.at[slot],
                              ssem.at[slot]).wait()

    def issue_dst(k, slot):
        pltpu.async_copy(dst_hbm.at[cstart + k], dst_t.at[slot], dsem.at[slot])

    def wait_dst(slot):
        pltpu.make_async_copy(dst_hbm.at[0], dst_t.at[slot],
                              dsem.at[slot]).wait()

    def issue_ga(k, islot, b):
        e0 = (cstart + k) * _C

        @pl.when(c == 0)
        def _():
            pltpu.async_copy(x0_hbm.at[src_t.at[islot]], gx.at[b], gsem.at[b])
            pltpu.async_copy(attr_hbm.at[pl.ds(e0, _C), pl.ds(0, _DH)],
                             at.at[b], asem.at[b])

        @pl.when(c == 1)
        def _():
            pltpu.async_copy(x1_hbm.at[src_t.at[islot]], gx.at[b], gsem.at[b])
            pltpu.async_copy(attr_hbm.at[pl.ds(e0, _C), pl.ds(_DH, _DH)],
                             at.at[b], asem.at[b])

    def wait_ga(b):
        pltpu.make_async_copy(x0_hbm.at[src_t.at[0]], gx.at[b],
                              gsem.at[b]).wait()
        pltpu.make_async_copy(attr_hbm.at[pl.ds(0, _C), pl.ds(0, _DH)],
                              at.at[b], asem.at[b]).wait()

    def wait_scat(b):
        pltpu.make_async_copy(pw.at[b], acc_sh.at[dst_t.at[0]],
                              scsem.at[b]).wait()

    for kk in range(_IBUF):
        issue_src(kk, kk)
    for kk in range(_NBUF):
        issue_dst(kk, kk)
    for kk in range(_NBUF):
        wait_src(kk)
        issue_ga(kk, kk, kk)

    def when(cond, fn):
        if isinstance(cond, bool):
            if cond:
                fn()
        else:
            pl.when(cond)(fn)

    def emit(k2, j):
        k = k2 * 4 + j
        b = j % 2
        bi = j

        wait_ga(b)
        when(k >= _NBUF, lambda: wait_scat(b))
        when(k + _NBUF < _CPT,
             lambda: issue_dst(k + _NBUF, (j + _NBUF) % _IBUF))

        @plsc.parallel_loop(0, _C, unroll=8)
        def _(i):
            for q in range(_DH // 16):
                g = gx[b, i, pl.ds(q * 16, 16)] + at[b, i, pl.ds(q * 16, 16)]
                m = jnp.maximum(g, 0.0)
                w = jnp.exp(m)
                pw[b, i, pl.ds(q * 16, 16)] = m * w
                pw[b, i, pl.ds(_DH + q * 16, 16)] = w

        wait_dst(bi)
        pltpu.async_copy(pw.at[b], acc_sh.at[dst_t.at[bi]],
                         scsem.at[b], add=True)

        when(k + _IBUF < _CPT, lambda: issue_src(k + _IBUF, bi))

        def _next_ga():
            wait_src((j + _NBUF) % _IBUF)
            issue_ga(k + _NBUF, (j + _NBUF) % _IBUF, b)

        when(k + _NBUF < _CPT, _next_ga)

    @pl.loop(0, _CPT // 4)
    def _(k2):
        for j in range(4):
            emit(k2, j)

    for j in range(_CPT % 4):
        emit(_CPT // 4, j)

    for b in range(_NBUF):
        wait_scat(b)
    plsc.subcore_barrier()

    for k in range(_RPT // _RB):
        r0 = rbase + k * _RB
        pltpu.sync_copy(acc_sh.at[pl.ds(r0, _RB)], stg)

        @pl.when(c == 0)
        def _():
            pltpu.sync_copy(stg, out0_hbm.at[pl.ds(r0, _RB)])

        @pl.when(c == 1)
        def _():
            pltpu.sync_copy(stg, out1_hbm.at[pl.ds(r0, _RB)])


@jax.jit
def _sc_aggregate(x0, x1, edge_attr, src2, dst2):
    mesh = plsc.VectorSubcoreMesh(core_axis_name="c", subcore_axis_name="s")
    acc_ty = jax.ShapeDtypeStruct((_N, _D), jnp.float32)
    run = pl.kernel(
        _sc_agg_body,
        out_type=[acc_ty, acc_ty],
        mesh=mesh,
        compiler_params=pltpu.CompilerParams(use_tc_tiling_on_sc=False),
        scratch_types=[
            pltpu.VMEM((_IBUF, _C), jnp.int32),
            pltpu.VMEM((_IBUF, _C), jnp.int32),
            pltpu.VMEM((_NBUF, _C, _DH), jnp.float32),
            pltpu.VMEM((_NBUF, _C, _DH), jnp.float32),
            pltpu.VMEM((_NBUF, _C, _D), jnp.float32),
            pltpu.VMEM((_RB, _D), jnp.float32),
            pltpu.VMEM_SHARED((_N, _D), jnp.float32),
            pltpu.SemaphoreType.DMA((_IBUF,)),
            pltpu.SemaphoreType.DMA((_IBUF,)),
            pltpu.SemaphoreType.DMA((_NBUF,)),
            pltpu.SemaphoreType.DMA((_NBUF,)),
            pltpu.SemaphoreType.DMA((_NBUF,)),
        ],
    )
    return run(x0, x1, edge_attr, src2, dst2)


def _mlp_body(acc0_ref, acc1_ref, x_ref, w1_ref, b1_ref, g_ref, be_ref,
              w2_ref, b2_ref, o_ref):
    num = jnp.concatenate([acc0_ref[:, :_DH], acc1_ref[:, :_DH]], axis=1)
    den = jnp.concatenate([acc0_ref[:, _DH:], acc1_ref[:, _DH:]], axis=1)
    out = (num + EPS * den) / (den + 1e-16) + x_ref[...]
    h = jnp.dot(out, w1_ref[...], preferred_element_type=jnp.float32) + b1_ref[...]
    mean = jnp.mean(h, axis=0, keepdims=True)
    var = jnp.mean((h - mean) * (h - mean), axis=0, keepdims=True)
    h = (h - mean) / jnp.sqrt(var + 1e-5) * g_ref[...] + be_ref[...]
    h = jnp.maximum(h, 0.0)
    o_ref[...] = jnp.dot(h, w2_ref[...], preferred_element_type=jnp.float32) + b2_ref[...]


def _mlp(acc0, acc1, x, p):
    return pl.pallas_call(
        _mlp_body,
        out_shape=jax.ShapeDtypeStruct((_N, _D), jnp.float32),
    )(acc0, acc1, x, p['W1'], p['b1'].reshape(1, -1), p['gamma'].reshape(1, -1),
      p['beta'].reshape(1, -1), p['W2'], p['b2'].reshape(1, -1))


def kernel(x, edge_index, edge_attr, params):
    src2 = edge_index[0].astype(jnp.int32).reshape(_NCHUNK, _C)
    dst2 = edge_index[1].astype(jnp.int32).reshape(_NCHUNK, _C)
    for p in params:
        x0 = x[:, :_DH]
        x1 = x[:, _DH:]
        acc0, acc1 = _sc_aggregate(x0, x1, edge_attr, src2, dst2)
        x = _mlp(acc0, acc1, x, p)
    return x


# speedup vs baseline: 1.1386x; 1.1386x over previous
import functools

import jax
import jax.numpy as jnp
from jax import lax
from jax.experimental import pallas as pl
from jax.experimental.pallas import tpu as pltpu
from jax.experimental.pallas import tpu_sc as plsc

EPS = 1e-7

_N = 10000
_E = 320000
_D = 128
_DH = 64
_C = 80
_NCHUNK = _E // _C
_NS = 16
_CPT = _NCHUNK // _NS
_NBUF = 2
_IBUF = 4
_RPT = _N // _NS
_RB = 25


def _sc_agg_body(x0_hbm, x1_hbm, attr_hbm, src_hbm, dst_hbm,
                 out0_hbm, out1_hbm,
                 src_t, dst_t, gx, at, pw, stg, acc_sh,
                 ssem, dsem, gsem, asem, scsem):
    c = lax.axis_index("c")
    s = lax.axis_index("s")

    @pl.loop(0, _RB)
    def _(i):
        for q in range(_D // 16):
            stg[i, pl.ds(q * 16, 16)] = jnp.zeros((16,), jnp.float32)

    rbase = s * _RPT
    for k in range(_RPT // _RB):
        pltpu.sync_copy(stg, acc_sh.at[pl.ds(rbase + k * _RB, _RB)])
    plsc.subcore_barrier()

    cstart = s * _CPT

    def issue_src(k, slot):
        pltpu.async_copy(src_hbm.at[cstart + k], src_t.at[slot], ssem.at[slot])

    def wait_src(slot):
        pltpu.make_async_copy(src_hbm.at[0], src_t.at[slot],
                              ssem.at[slot]).wait()

    def issue_dst(k, slot):
        pltpu.async_copy(dst_hbm.at[cstart + k], dst_t.at[slot], dsem.at[slot])

    def wait_dst(slot):
        pltpu.make_async_copy(dst_hbm.at[0], dst_t.at[slot],
                              dsem.at[slot]).wait()

    def issue_ga(k, islot, b):
        e0 = (cstart + k) * _C

        @pl.when(c == 0)
        def _():
            pltpu.async_copy(x0_hbm.at[src_t.at[islot]], gx.at[b], gsem.at[b])
            pltpu.async_copy(attr_hbm.at[pl.ds(e0, _C), pl.ds(0, _DH)],
                             at.at[b], asem.at[b])

        @pl.when(c == 1)
        def _():
            pltpu.async_copy(x1_hbm.at[src_t.at[islot]], gx.at[b], gsem.at[b])
            pltpu.async_copy(attr_hbm.at[pl.ds(e0, _C), pl.ds(_DH, _DH)],
                             at.at[b], asem.at[b])

    def wait_ga(b):
        pltpu.make_async_copy(x0_hbm.at[src_t.at[0]], gx.at[b],
                              gsem.at[b]).wait()
        pltpu.make_async_copy(attr_hbm.at[pl.ds(0, _C), pl.ds(0, _DH)],
                              at.at[b], asem.at[b]).wait()

    def wait_scat(b):
        pltpu.make_async_copy(pw.at[b], acc_sh.at[dst_t.at[0]],
                              scsem.at[b]).wait()

    for kk in range(_IBUF):
        issue_src(kk, kk)
    for kk in range(_NBUF):
        issue_dst(kk, kk)
    for kk in range(_NBUF):
        wait_src(kk)
        issue_ga(kk, kk, kk)

    def when(cond, fn):
        if isinstance(cond, bool):
            if cond:
                fn()
        else:
            pl.when(cond)(fn)

    def emit(k2, j):
        k = k2 * 4 + j
        b = j % 2
        bi = j

        wait_ga(b)
        when(k >= _NBUF, lambda: wait_scat(b))
        when(k + _NBUF < _CPT,
             lambda: issue_dst(k + _NBUF, (j + _NBUF) % _IBUF))

        @plsc.parallel_loop(0, _C, unroll=2)
        def _(i):
            for q in range(_DH // 16):
                g = gx[b, i, pl.ds(q * 16, 16)] + at[b, i, pl.ds(q * 16, 16)]
                m = jnp.maximum(g, 0.0)
                w = jnp.exp(m)
                pw[b, i, pl.ds(q * 16, 16)] = m * w
                pw[b, i, pl.ds(_DH + q * 16, 16)] = w

        wait_dst(bi)
        pltpu.async_copy(pw.at[b], acc_sh.at[dst_t.at[bi]],
                         scsem.at[b], add=True)

        when(k + _IBUF < _CPT, lambda: issue_src(k + _IBUF, bi))

        def _next_ga():
            wait_src((j + _NBUF) % _IBUF)
            issue_ga(k + _NBUF, (j + _NBUF) % _IBUF, b)

        when(k + _NBUF < _CPT, _next_ga)

    @pl.loop(0, _CPT // 4)
    def _(k2):
        for j in range(4):
            emit(k2, j)

    for j in range(_CPT % 4):
        emit(_CPT // 4, j)

    for b in range(_NBUF):
        wait_scat(b)
    plsc.subcore_barrier()

    for k in range(_RPT // _RB):
        r0 = rbase + k * _RB
        pltpu.sync_copy(acc_sh.at[pl.ds(r0, _RB)], stg)

        @pl.when(c == 0)
        def _():
            pltpu.sync_copy(stg, out0_hbm.at[pl.ds(r0, _RB)])

        @pl.when(c == 1)
        def _():
            pltpu.sync_copy(stg, out1_hbm.at[pl.ds(r0, _RB)])


@jax.jit
def _sc_aggregate(x0, x1, edge_attr, src2, dst2):
    mesh = plsc.VectorSubcoreMesh(core_axis_name="c", subcore_axis_name="s")
    acc_ty = jax.ShapeDtypeStruct((_N, _D), jnp.float32)
    run = pl.kernel(
        _sc_agg_body,
        out_type=[acc_ty, acc_ty],
        mesh=mesh,
        compiler_params=pltpu.CompilerParams(use_tc_tiling_on_sc=False),
        scratch_types=[
            pltpu.VMEM((_IBUF, _C), jnp.int32),
            pltpu.VMEM((_IBUF, _C), jnp.int32),
            pltpu.VMEM((_NBUF, _C, _DH), jnp.float32),
            pltpu.VMEM((_NBUF, _C, _DH), jnp.float32),
            pltpu.VMEM((_NBUF, _C, _D), jnp.float32),
            pltpu.VMEM((_RB, _D), jnp.float32),
            pltpu.VMEM_SHARED((_N, _D), jnp.float32),
            pltpu.SemaphoreType.DMA((_IBUF,)),
            pltpu.SemaphoreType.DMA((_IBUF,)),
            pltpu.SemaphoreType.DMA((_NBUF,)),
            pltpu.SemaphoreType.DMA((_NBUF,)),
            pltpu.SemaphoreType.DMA((_NBUF,)),
        ],
    )
    return run(x0, x1, edge_attr, src2, dst2)


def _mlp_body(acc0_ref, acc1_ref, x_ref, w1_ref, b1_ref, g_ref, be_ref,
              w2_ref, b2_ref, o_ref):
    num = jnp.concatenate([acc0_ref[:, :_DH], acc1_ref[:, :_DH]], axis=1)
    den = jnp.concatenate([acc0_ref[:, _DH:], acc1_ref[:, _DH:]], axis=1)
    out = (num + EPS * den) / (den + 1e-16) + x_ref[...]
    h = jnp.dot(out, w1_ref[...], preferred_element_type=jnp.float32) + b1_ref[...]
    mean = jnp.mean(h, axis=0, keepdims=True)
    var = jnp.mean((h - mean) * (h - mean), axis=0, keepdims=True)
    h = (h - mean) / jnp.sqrt(var + 1e-5) * g_ref[...] + be_ref[...]
    h = jnp.maximum(h, 0.0)
    o_ref[...] = jnp.dot(h, w2_ref[...], preferred_element_type=jnp.float32) + b2_ref[...]


def _mlp(acc0, acc1, x, p):
    return pl.pallas_call(
        _mlp_body,
        out_shape=jax.ShapeDtypeStruct((_N, _D), jnp.float32),
    )(acc0, acc1, x, p['W1'], p['b1'].reshape(1, -1), p['gamma'].reshape(1, -1),
      p['beta'].reshape(1, -1), p['W2'], p['b2'].reshape(1, -1))


def kernel(x, edge_index, edge_attr, params):
    src2 = edge_index[0].astype(jnp.int32).reshape(_NCHUNK, _C)
    dst2 = edge_index[1].astype(jnp.int32).reshape(_NCHUNK, _C)
    for p in params:
        x0 = x[:, :_DH]
        x1 = x[:, _DH:]
        acc0, acc1 = _sc_aggregate(x0, x1, edge_attr, src2, dst2)
        x = _mlp(acc0, acc1, x, p)
    return x


# speedup vs baseline: 1.1622x; 1.0207x over previous
import functools

import jax
import jax.numpy as jnp
from jax import lax
from jax.experimental import pallas as pl
from jax.experimental.pallas import tpu as pltpu
from jax.experimental.pallas import tpu_sc as plsc

EPS = 1e-7

_N = 10000
_E = 320000
_D = 128
_DH = 64
_C = 80
_NCHUNK = _E // _C
_NS = 16
_CPT = _NCHUNK // _NS
_NBUF = 2
_IBUF = 4
_RPT = _N // _NS
_RB = 25


def _sc_agg_body(x0_hbm, x1_hbm, attr_hbm, src_hbm, dst_hbm,
                 out0_hbm, out1_hbm,
                 src_t, dst_t, gx, at, pw, stg, acc_sh,
                 ssem, dsem, gsem, asem, scsem, hsem):
    c = lax.axis_index("c")
    s = lax.axis_index("s")

    @pl.loop(0, _RB)
    def _(i):
        for q in range(_D // 16):
            stg[0, i, pl.ds(q * 16, 16)] = jnp.zeros((16,), jnp.float32)

    rbase = s * _RPT
    for k in range(_RPT // _RB):
        pltpu.sync_copy(stg.at[0], acc_sh.at[pl.ds(rbase + k * _RB, _RB)])
    plsc.subcore_barrier()

    cstart = s * _CPT

    def issue_src(k, slot):
        pltpu.async_copy(src_hbm.at[cstart + k], src_t.at[slot], ssem.at[slot])

    def wait_src(slot):
        pltpu.make_async_copy(src_hbm.at[0], src_t.at[slot],
                              ssem.at[slot]).wait()

    def issue_dst(k, slot):
        pltpu.async_copy(dst_hbm.at[cstart + k], dst_t.at[slot], dsem.at[slot])

    def wait_dst(slot):
        pltpu.make_async_copy(dst_hbm.at[0], dst_t.at[slot],
                              dsem.at[slot]).wait()

    def issue_ga(k, islot, b):
        e0 = (cstart + k) * _C

        @pl.when(c == 0)
        def _():
            pltpu.async_copy(x0_hbm.at[src_t.at[islot]], gx.at[b], gsem.at[b])
            pltpu.async_copy(attr_hbm.at[pl.ds(e0, _C), pl.ds(0, _DH)],
                             at.at[b], asem.at[b])

        @pl.when(c == 1)
        def _():
            pltpu.async_copy(x1_hbm.at[src_t.at[islot]], gx.at[b], gsem.at[b])
            pltpu.async_copy(attr_hbm.at[pl.ds(e0, _C), pl.ds(_DH, _DH)],
                             at.at[b], asem.at[b])

    def wait_ga(b):
        pltpu.make_async_copy(x0_hbm.at[src_t.at[0]], gx.at[b],
                              gsem.at[b]).wait()
        pltpu.make_async_copy(attr_hbm.at[pl.ds(0, _C), pl.ds(0, _DH)],
                              at.at[b], asem.at[b]).wait()

    def wait_scat(b):
        pltpu.make_async_copy(pw.at[b], acc_sh.at[dst_t.at[0]],
                              scsem.at[b]).wait()

    for kk in range(_IBUF):
        issue_src(kk, kk)
    for kk in range(_NBUF):
        issue_dst(kk, kk)
    for kk in range(_NBUF):
        wait_src(kk)
        issue_ga(kk, kk, kk)

    def when(cond, fn):
        if isinstance(cond, bool):
            if cond:
                fn()
        else:
            pl.when(cond)(fn)

    def emit(k2, j):
        k = k2 * 4 + j
        b = j % 2
        bi = j

        wait_ga(b)
        when(k >= _NBUF, lambda: wait_scat(b))
        when(k + _NBUF < _CPT,
             lambda: issue_dst(k + _NBUF, (j + _NBUF) % _IBUF))

        @plsc.parallel_loop(0, _C, unroll=4)
        def _(i):
            for q in range(_DH // 16):
                g = gx[b, i, pl.ds(q * 16, 16)] + at[b, i, pl.ds(q * 16, 16)]
                m = jnp.maximum(g, 0.0)
                w = jnp.exp(m)
                pw[b, i, pl.ds(q * 16, 16)] = m * w
                pw[b, i, pl.ds(_DH + q * 16, 16)] = w

        wait_dst(bi)
        pltpu.async_copy(pw.at[b], acc_sh.at[dst_t.at[bi]],
                         scsem.at[b], add=True)

        when(k + _IBUF < _CPT, lambda: issue_src(k + _IBUF, bi))

        def _next_ga():
            wait_src((j + _NBUF) % _IBUF)
            issue_ga(k + _NBUF, (j + _NBUF) % _IBUF, b)

        when(k + _NBUF < _CPT, _next_ga)

    @pl.loop(0, _CPT // 4)
    def _(k2):
        for j in range(4):
            emit(k2, j)

    for j in range(_CPT % 4):
        emit(_CPT // 4, j)

    for b in range(_NBUF):
        wait_scat(b)
    plsc.subcore_barrier()

    def dump(out_hbm):
        for k in range(_RPT // _RB):
            b = k % 2
            if k >= 2:
                pltpu.make_async_copy(stg.at[b], out_hbm.at[pl.ds(0, _RB)],
                                      hsem.at[b]).wait()
            pltpu.sync_copy(acc_sh.at[pl.ds(rbase + k * _RB, _RB)], stg.at[b])
            pltpu.async_copy(stg.at[b], out_hbm.at[pl.ds(rbase + k * _RB, _RB)],
                             hsem.at[b])
        for b in range(2):
            pltpu.make_async_copy(stg.at[b], out_hbm.at[pl.ds(0, _RB)],
                                  hsem.at[b]).wait()

    @pl.when(c == 0)
    def _():
        dump(out0_hbm)

    @pl.when(c == 1)
    def _():
        dump(out1_hbm)


@jax.jit
def _sc_aggregate(x0, x1, edge_attr, src2, dst2):
    mesh = plsc.VectorSubcoreMesh(core_axis_name="c", subcore_axis_name="s")
    acc_ty = jax.ShapeDtypeStruct((_N, _D), jnp.float32)
    run = pl.kernel(
        _sc_agg_body,
        out_type=[acc_ty, acc_ty],
        mesh=mesh,
        compiler_params=pltpu.CompilerParams(use_tc_tiling_on_sc=False),
        scratch_types=[
            pltpu.VMEM((_IBUF, _C), jnp.int32),
            pltpu.VMEM((_IBUF, _C), jnp.int32),
            pltpu.VMEM((_NBUF, _C, _DH), jnp.float32),
            pltpu.VMEM((_NBUF, _C, _DH), jnp.float32),
            pltpu.VMEM((_NBUF, _C, _D), jnp.float32),
            pltpu.VMEM((2, _RB, _D), jnp.float32),
            pltpu.VMEM_SHARED((_N, _D), jnp.float32),
            pltpu.SemaphoreType.DMA((_IBUF,)),
            pltpu.SemaphoreType.DMA((_IBUF,)),
            pltpu.SemaphoreType.DMA((_NBUF,)),
            pltpu.SemaphoreType.DMA((_NBUF,)),
            pltpu.SemaphoreType.DMA((_NBUF,)),
            pltpu.SemaphoreType.DMA((2,)),
        ],
    )
    return run(x0, x1, edge_attr, src2, dst2)


def _mlp_body(acc0_ref, acc1_ref, x_ref, w1_ref, b1_ref, g_ref, be_ref,
              w2_ref, b2_ref, o_ref):
    num = jnp.concatenate([acc0_ref[:, :_DH], acc1_ref[:, :_DH]], axis=1)
    den = jnp.concatenate([acc0_ref[:, _DH:], acc1_ref[:, _DH:]], axis=1)
    out = (num + EPS * den) / (den + 1e-16) + x_ref[...]
    h = jnp.dot(out, w1_ref[...], preferred_element_type=jnp.float32) + b1_ref[...]
    mean = jnp.mean(h, axis=0, keepdims=True)
    var = jnp.mean((h - mean) * (h - mean), axis=0, keepdims=True)
    h = (h - mean) / jnp.sqrt(var + 1e-5) * g_ref[...] + be_ref[...]
    h = jnp.maximum(h, 0.0)
    o_ref[...] = jnp.dot(h, w2_ref[...], preferred_element_type=jnp.float32) + b2_ref[...]


def _mlp(acc0, acc1, x, p):
    return pl.pallas_call(
        _mlp_body,
        out_shape=jax.ShapeDtypeStruct((_N, _D), jnp.float32),
    )(acc0, acc1, x, p['W1'], p['b1'].reshape(1, -1), p['gamma'].reshape(1, -1),
      p['beta'].reshape(1, -1), p['W2'], p['b2'].reshape(1, -1))


def kernel(x, edge_index, edge_attr, params):
    src2 = edge_index[0].astype(jnp.int32).reshape(_NCHUNK, _C)
    dst2 = edge_index[1].astype(jnp.int32).reshape(_NCHUNK, _C)
    for p in params:
        x0 = x[:, :_DH]
        x1 = x[:, _DH:]
        acc0, acc1 = _sc_aggregate(x0, x1, edge_attr, src2, dst2)
        x = _mlp(acc0, acc1, x, p)
    return x
